# baseline (speedup 1.0000x reference)
import jax
import jax.numpy as jnp
from jax import lax
from jax.experimental import pallas as pl
from jax.experimental.pallas import tpu as pltpu
from jax.experimental.pallas import tpu_sc as plsc

N = 10000
E = 160000
D = 256
A = 128
NP = 10240
NC = 2
NS = 16
DH = D // NC

NP2 = 10112
STRIPE = NP2 // NS

EPMAX = 172032
CH1 = 128
EPP1 = EPMAX // (NC * NS)
NCH1 = EPP1 // CH1
CH2 = 48
EPT2 = EPMAX // NS
NCH2 = EPT2 // CH2


def _mm_body(x_ref, w_ref, o_ref):
    o_ref[...] = jnp.dot(x_ref[...], w_ref[...],
                         preferred_element_type=jnp.float32)


def _matmul(x, w, bm):
    m, k = x.shape
    n = w.shape[1]
    return pl.pallas_call(
        _mm_body,
        grid=(m // bm,),
        in_specs=[
            pl.BlockSpec((bm, k), lambda i: (i, 0)),
            pl.BlockSpec((k, n), lambda i: (0, 0)),
        ],
        out_specs=pl.BlockSpec((bm, n), lambda i: (i, 0)),
        out_shape=jax.ShapeDtypeStruct((m, n), jnp.float32),
    )(x, w)


def _mm_bias_body(x_ref, w_ref, b_ref, o_ref):
    o_ref[...] = jnp.dot(x_ref[...], w_ref[...],
                         preferred_element_type=jnp.float32) + b_ref[...]


def _matmul_bias(x, w, b):
    m, k = x.shape
    n = w.shape[1]
    return pl.pallas_call(
        _mm_bias_body,
        grid=(1,),
        in_specs=[
            pl.BlockSpec((m, k), lambda i: (0, 0)),
            pl.BlockSpec((k, n), lambda i: (0, 0)),
            pl.BlockSpec((1, n), lambda i: (0, 0)),
        ],
        out_specs=pl.BlockSpec((m, n), lambda i: (0, 0)),
        out_shape=jax.ShapeDtypeStruct((m, n), jnp.float32),
    )(x, w, b.reshape(1, n))


def _final_body(agg_ref, w2_ref, o_ref):
    a = agg_ref[...]
    w2 = w2_ref[...]
    o_ref[...] = (jnp.dot(a[0], w2[0], preferred_element_type=jnp.float32)
                  + jnp.dot(a[1], w2[1], preferred_element_type=jnp.float32))


def _final_matmul(agg2, w2, bm):
    m = agg2.shape[1]
    return pl.pallas_call(
        _final_body,
        grid=(m // bm,),
        in_specs=[
            pl.BlockSpec((NC, bm, DH), lambda i: (0, i, 0)),
            pl.BlockSpec((NC, DH, D), lambda i: (0, 0, 0)),
        ],
        out_specs=pl.BlockSpec((bm, D), lambda i: (i, 0)),
        out_shape=jax.ShapeDtypeStruct((m, D), jnp.float32),
    )(agg2, w2)


def _alpha_groups(ngrp, hp_v, rp_v, qp_v, wa_v, wb_v, al_v):
    eiota = lax.iota(jnp.int32, 16)

    def _egrp(v, carry):
        sgrp = jnp.zeros((16,), jnp.float32)
        for e0 in range(16):
            e = 16 * v + e0
            acc = jnp.zeros((16,), jnp.float32)
            for j in range(A // 16):
                sl = pl.ds(16 * j, 16)
                pre = hp_v[e, sl] + rp_v[e, sl] + qp_v[e, sl]
                acc = acc + jnp.maximum(pre, 0.0) * wa_v[sl]
            for sh in (8, 4, 2, 1):
                acc = acc + acc.at[eiota ^ sh].get(mode="promise_in_bounds")
            sgrp = jnp.where(eiota == e0, acc, sgrp)
        x = sgrp + wb_v[...]
        al_v[pl.ds(16 * v, 16)] = 1.0 / (1.0 + jnp.exp(-x))
        return carry

    lax.fori_loop(0, ngrp, _egrp, 0)


def _sc_alpha(idx3_h, hp_h, rp_h, qp_h, wa_h, wb_h, al_h,
              i3a_v, i3b_v, hpa_v, hpb_v, rpa_v, rpb_v, qpa_v, qpb_v,
              ala_v, alb_v, wa_v, wb_v, semga, semgb):
    c = lax.axis_index("c")
    s = lax.axis_index("s")
    w = s * NC + c
    tbase = w * EPP1

    pltpu.sync_copy(wa_h, wa_v)
    pltpu.sync_copy(wb_h, wb_v)

    i3 = (i3a_v, i3b_v)
    hpv = (hpa_v, hpb_v)
    rpv = (rpa_v, rpb_v)
    qpv = (qpa_v, qpb_v)
    alv = (ala_v, alb_v)
    semg = (semga, semgb)

    def _issue(g, b):
        pltpu.sync_copy(idx3_h.at[:, pl.ds(tbase + g * CH1, CH1)], i3[b])
        pltpu.async_copy(hp_h.at[i3[b].at[0]], hpv[b], semg[b])
        pltpu.async_copy(rp_h.at[i3[b].at[1]], rpv[b], semg[b])
        pltpu.async_copy(qp_h.at[i3[b].at[2]], qpv[b], semg[b])

    def _wait_g(b):
        pltpu.make_async_copy(hp_h.at[i3[b].at[0]], hpv[b], semg[b]).wait()
        pltpu.make_async_copy(rp_h.at[i3[b].at[1]], rpv[b], semg[b]).wait()
        pltpu.make_async_copy(qp_h.at[i3[b].at[2]], qpv[b], semg[b]).wait()

    _issue(0, 0)

    def _pair(gp, carry):
        for b in range(2):
            g = 2 * gp + b

            @pl.when(g + 1 < NCH1)
            def _():
                _issue(g + 1, 1 - b)

            _wait_g(b)
            _alpha_groups(CH1 // 16, hpv[b], rpv[b], qpv[b], wa_v, wb_v,
                          alv[b])
            pltpu.sync_copy(alv[b], al_h.at[pl.ds(tbase + g * CH1, CH1)])
        return carry

    lax.fori_loop(0, NCH1 // 2, _pair, 0)


def _sc_alpha_call(idx3, hp, rp, qp, wa, wb16):
    mesh = plsc.VectorSubcoreMesh(core_axis_name="c", subcore_axis_name="s",
                                  num_cores=NC, num_subcores=NS)
    f = pl.kernel(
        _sc_alpha,
        out_type=jax.ShapeDtypeStruct((EPMAX,), jnp.float32),
        mesh=mesh,
        scratch_types=[
            pltpu.VMEM((3, CH1), jnp.int32),
            pltpu.VMEM((3, CH1), jnp.int32),
            pltpu.VMEM((CH1, A), jnp.float32),
            pltpu.VMEM((CH1, A), jnp.float32),
            pltpu.VMEM((CH1, A), jnp.float32),
            pltpu.VMEM((CH1, A), jnp.float32),
            pltpu.VMEM((CH1, A), jnp.float32),
            pltpu.VMEM((CH1, A), jnp.float32),
            pltpu.VMEM((CH1,), jnp.float32),
            pltpu.VMEM((CH1,), jnp.float32),
            pltpu.VMEM((A,), jnp.float32),
            pltpu.VMEM((16,), jnp.float32),
            pltpu.SemaphoreType.DMA,
            pltpu.SemaphoreType.DMA,
        ],
    )
    return f(idx3, hp, rp, qp, wa, wb16)


def _sc_msg(sub_h, rel_h, obj_h, al_h, hid_h, rla_h, out_h,
            suba_v, subb_v, rela_v, relb_v, ala_v, alb_v,
            obj0_v, obj1_v, obj2_v, obj3_v,
            hsa_v, hsb_v, hra_v, hrb_v, msga_v, msgb_v,
            agg_sh, semia, semib, semga, semgb, sems):
    c = lax.axis_index("c")
    s = lax.axis_index("s")
    tbase = s * EPT2

    subv = (suba_v, subb_v)
    relv = (rela_v, relb_v)
    alv = (ala_v, alb_v)
    objv = (obj0_v, obj1_v, obj2_v, obj3_v)
    hsv = (hsa_v, hsb_v)
    hrv = (hra_v, hrb_v)
    msgv = (msga_v, msgb_v)
    semi = (semia, semib)
    semg = (semga, semgb)

    z16 = jnp.zeros((16,), jnp.float32)

    def _zrow(i, carry):
        for j in range(DH // 16):
            msga_v[i, 16 * j:16 * (j + 1)] = z16
        return carry

    lax.fori_loop(0, CH2, _zrow, 0)
    nfull = STRIPE // CH2
    rem = STRIPE - nfull * CH2
    for k in range(nfull):
        pltpu.sync_copy(msga_v, agg_sh.at[pl.ds(s * STRIPE + k * CH2, CH2)])
    if rem:
        pltpu.sync_copy(msga_v.at[pl.ds(0, rem)],
                        agg_sh.at[pl.ds(s * STRIPE + nfull * CH2, rem)])
    plsc.subcore_barrier()

    def _issue_idx(g, b2, b4):
        base = tbase + g * CH2
        pltpu.async_copy(sub_h.at[pl.ds(base, CH2)], subv[b2], semi[b2])
        pltpu.async_copy(rel_h.at[pl.ds(base, CH2)], relv[b2], semi[b2])
        pltpu.async_copy(al_h.at[pl.ds(base, CH2)],
                         alv[b2].at[pl.ds(0, CH2)], semi[b2])
        pltpu.async_copy(obj_h.at[pl.ds(base, CH2)], objv[b4], semi[b2])

    def _wait_idx(b2, b4):
        pltpu.make_async_copy(sub_h.at[pl.ds(tbase, CH2)], subv[b2],
                              semi[b2]).wait()
        pltpu.make_async_copy(rel_h.at[pl.ds(tbase, CH2)], relv[b2],
                              semi[b2]).wait()
        pltpu.make_async_copy(al_h.at[pl.ds(tbase, CH2)],
                              alv[b2].at[pl.ds(0, CH2)], semi[b2]).wait()
        pltpu.make_async_copy(obj_h.at[pl.ds(tbase, CH2)], objv[b4],
                              semi[b2]).wait()

    def _issue_g(b2):
        pltpu.async_copy(hid_h.at[c].at[subv[b2]], hsv[b2], semg[b2])
        pltpu.async_copy(rla_h.at[c].at[relv[b2]], hrv[b2], semg[b2])

    def _wait_g(b2):
        pltpu.make_async_copy(hid_h.at[c].at[subv[b2]], hsv[b2],
                              semg[b2]).wait()
        pltpu.make_async_copy(rla_h.at[c].at[relv[b2]], hrv[b2],
                              semg[b2]).wait()

    def _wait_s(b2, b4):
        pltpu.make_async_copy(msgv[b2], agg_sh.at[objv[b4]], sems).wait()

    _issue_idx(0, 0, 0)
    _issue_idx(1, 1, 1)
    _wait_idx(0, 0)
    _issue_g(0)

    def _quad(qp, carry):
        for b in range(4):
            g = 4 * qp + b
            b2 = b % 2

            @pl.when(g >= 2)
            def _():
                _wait_s(b2, b)

            _wait_g(b2)

            @pl.when(g + 1 < NCH2)
            def _():
                _wait_idx(1 - b2, (b + 1) % 4)
                _issue_g(1 - b2)

            def _emsg(e, carry2):
                a = alv[b2][pl.ds(e, 16)][0]
                for j in range(DH // 16):
                    sl = pl.ds(16 * j, 16)
                    msgv[b2][e, sl] = hsv[b2][e, sl] * hrv[b2][e, sl] * a
                return carry2

            lax.fori_loop(0, CH2, _emsg, 0)

            @pl.when(g + 2 < NCH2)
            def _():
                _issue_idx(g + 2, b2, (b + 2) % 4)

            pltpu.async_copy(msgv[b2], agg_sh.at[objv[b]], sems, add=True)
        return carry

    lax.fori_loop(0, NCH2 // 4, _quad, 0)
    _wait_s(0, 2)
    _wait_s(1, 3)

    plsc.subcore_barrier()
    pltpu.sync_copy(agg_sh.at[pl.ds(s * STRIPE, STRIPE)],
                    out_h.at[c, pl.ds(s * STRIPE, STRIPE)])


def _sc_msg_call(sub, rel, obj, alpha, hid2, rla2):
    mesh = plsc.VectorSubcoreMesh(core_axis_name="c", subcore_axis_name="s",
                                  num_cores=NC, num_subcores=NS)
    f = pl.kernel(
        _sc_msg,
        out_type=jax.ShapeDtypeStruct((NC, NP, DH), jnp.float32),
        mesh=mesh,
        scratch_types=[
            pltpu.VMEM((CH2,), jnp.int32),
            pltpu.VMEM((CH2,), jnp.int32),
            pltpu.VMEM((CH2,), jnp.int32),
            pltpu.VMEM((CH2,), jnp.int32),
            pltpu.VMEM((CH2 + 16,), jnp.float32),
            pltpu.VMEM((CH2 + 16,), jnp.float32),
            pltpu.VMEM((CH2,), jnp.int32),
            pltpu.VMEM((CH2,), jnp.int32),
            pltpu.VMEM((CH2,), jnp.int32),
            pltpu.VMEM((CH2,), jnp.int32),
            pltpu.VMEM((CH2, DH), jnp.float32),
            pltpu.VMEM((CH2, DH), jnp.float32),
            pltpu.VMEM((CH2, DH), jnp.float32),
            pltpu.VMEM((CH2, DH), jnp.float32),
            pltpu.VMEM((CH2, DH), jnp.float32),
            pltpu.VMEM((CH2, DH), jnp.float32),
            pltpu.VMEM_SHARED((NP2, DH), jnp.float32),
            pltpu.SemaphoreType.DMA,
            pltpu.SemaphoreType.DMA,
            pltpu.SemaphoreType.DMA,
            pltpu.SemaphoreType.DMA,
            pltpu.SemaphoreType.DMA,
        ],
    )
    return f(sub, rel, obj, alpha, hid2, rla2)


def kernel(q_sub, q_rel, r_idx, hidden, edges, n_node, rela_embed,
           Ws_attn, Wr_attn, Wqr_attn_W, Wqr_attn_b,
           w_alpha_W, w_alpha_b, W_h):
    sub = jnp.pad(edges[:, 0].astype(jnp.int32), (0, EPMAX - E))
    rel = jnp.pad(edges[:, 1].astype(jnp.int32), (0, EPMAX - E))
    obj = jnp.pad(edges[:, 2].astype(jnp.int32), (0, EPMAX - E),
                  constant_values=NP2 - 1)
    rid = jnp.pad(r_idx.astype(jnp.int32), (0, EPMAX - E))
    idx3 = jnp.stack([sub, rel, rid])

    hid_p = jnp.pad(hidden, ((0, NP - N), (0, 0)))
    rla_p = jnp.pad(rela_embed, ((0, NP - rela_embed.shape[0]), (0, 0)))

    hp = _matmul(hid_p, Ws_attn, 512)
    rp = _matmul(rla_p, Wr_attn, 512)
    qsel = jnp.take(rela_embed, q_rel, axis=0)
    qp = _matmul_bias(qsel, Wqr_attn_W, Wqr_attn_b)

    wa = w_alpha_W[:, 0]
    wb16 = jnp.broadcast_to(w_alpha_b, (16,)).astype(jnp.float32)

    alpha = _sc_alpha_call(idx3, hp, rp, qp, wa, wb16)

    hid2 = hid_p.reshape(NP, NC, DH).transpose(1, 0, 2)
    rla2 = rla_p.reshape(NP, NC, DH).transpose(1, 0, 2)

    agg2 = _sc_msg_call(sub, rel, obj, alpha, hid2, rla2)

    w2 = W_h.reshape(NC, DH, D)
    out = _final_matmul(agg2, w2, 512)
    return out[:N]


# speedup vs baseline: 1.3599x; 1.3599x over previous
import jax
import jax.numpy as jnp
from jax import lax
from jax.experimental import pallas as pl
from jax.experimental.pallas import tpu as pltpu
from jax.experimental.pallas import tpu_sc as plsc

N = 10000
E = 160000
D = 256
A = 128
B = 256
NP = 10240
NC = 2
NS = 16
DH = D // NC

NP2 = 10112
STRIPE = NP2 // NS

EPMAX = 165888
CH1 = 96
EPP1 = EPMAX // (NC * NS)
NCH1 = EPP1 // CH1
CH2 = 48
EPT2 = EPMAX // NS
NCH2 = EPT2 // CH2


def _mm_body(x_ref, w_ref, o_ref):
    o_ref[...] = jnp.dot(x_ref[...], w_ref[...],
                         preferred_element_type=jnp.float32)


def _matmul(x, w, bm):
    m, k = x.shape
    n = w.shape[1]
    return pl.pallas_call(
        _mm_body,
        grid=(m // bm,),
        in_specs=[
            pl.BlockSpec((bm, k), lambda i: (i, 0)),
            pl.BlockSpec((k, n), lambda i: (0, 0)),
        ],
        out_specs=pl.BlockSpec((bm, n), lambda i: (i, 0)),
        out_shape=jax.ShapeDtypeStruct((m, n), jnp.float32),
    )(x, w)


def _mm_bias_body(x_ref, w_ref, b_ref, o_ref):
    o_ref[...] = jnp.dot(x_ref[...], w_ref[...],
                         preferred_element_type=jnp.float32) + b_ref[...]


def _matmul_bias(x, w, b):
    m, k = x.shape
    n = w.shape[1]
    return pl.pallas_call(
        _mm_bias_body,
        grid=(1,),
        in_specs=[
            pl.BlockSpec((m, k), lambda i: (0, 0)),
            pl.BlockSpec((k, n), lambda i: (0, 0)),
            pl.BlockSpec((1, n), lambda i: (0, 0)),
        ],
        out_specs=pl.BlockSpec((m, n), lambda i: (0, 0)),
        out_shape=jax.ShapeDtypeStruct((m, n), jnp.float32),
    )(x, w, b.reshape(1, n))


def _final_body(agg_ref, w2_ref, o_ref):
    a = agg_ref[...]
    w2 = w2_ref[...]
    o_ref[...] = (jnp.dot(a[0], w2[0], preferred_element_type=jnp.float32)
                  + jnp.dot(a[1], w2[1], preferred_element_type=jnp.float32))


def _final_matmul(agg2, w2, bm):
    m = agg2.shape[1]
    return pl.pallas_call(
        _final_body,
        grid=(m // bm,),
        in_specs=[
            pl.BlockSpec((NC, bm, DH), lambda i: (0, i, 0)),
            pl.BlockSpec((NC, DH, D), lambda i: (0, 0, 0)),
        ],
        out_specs=pl.BlockSpec((bm, D), lambda i: (i, 0)),
        out_shape=jax.ShapeDtypeStruct((m, D), jnp.float32),
    )(agg2, w2)


def _alpha_groups(ngrp, hp_v, rp_v, qp_v, wa_v, wb_v, al_v):
    eiota = lax.iota(jnp.int32, 16)

    def _egrp(v, carry):
        sgrp = jnp.zeros((16,), jnp.float32)
        for e0 in range(16):
            e = 16 * v + e0
            acc = jnp.zeros((16,), jnp.float32)
            for j in range(A // 16):
                sl = pl.ds(16 * j, 16)
                pre = hp_v[e, sl] + rp_v[e, sl] + qp_v[e, sl]
                acc = acc + jnp.maximum(pre, 0.0) * wa_v[sl]
            for sh in (8, 4, 2, 1):
                acc = acc + acc.at[eiota ^ sh].get(mode="promise_in_bounds")
            sgrp = jnp.where(eiota == e0, acc, sgrp)
        x = sgrp + wb_v[...]
        al_v[pl.ds(16 * v, 16)] = 1.0 / (1.0 + jnp.exp(-x))
        return carry

    lax.fori_loop(0, ngrp, _egrp, 0)


def _sc_alpha(sub_h, rel_h, rid_h, hp_h, rp_h, qp_h, wa_h, wb_h, al_h,
              suba_v, subb_v, rela_v, relb_v, rida_v, ridb_v,
              hpa_v, hpb_v, rpa_v, rpb_v,
              qp_v, ala_v, alb_v, wa_v, wb_v, semga, semgb):
    c = lax.axis_index("c")
    s = lax.axis_index("s")
    w = s * NC + c
    tbase = w * EPP1

    pltpu.sync_copy(wa_h, wa_v)
    pltpu.sync_copy(wb_h, wb_v)
    pltpu.sync_copy(qp_h, qp_v)

    subv = (suba_v, subb_v)
    relv = (rela_v, relb_v)
    ridv = (rida_v, ridb_v)
    hpv = (hpa_v, hpb_v)
    rpv = (rpa_v, rpb_v)
    alv = (ala_v, alb_v)
    semg = (semga, semgb)

    def _issue(g, b):
        base = tbase + g * CH1
        pltpu.sync_copy(sub_h.at[pl.ds(base, CH1)], subv[b])
        pltpu.sync_copy(rel_h.at[pl.ds(base, CH1)], relv[b])
        pltpu.sync_copy(rid_h.at[pl.ds(base, CH1)],
                        ridv[b].at[pl.ds(0, CH1)])
        pltpu.async_copy(hp_h.at[subv[b]], hpv[b], semg[b])
        pltpu.async_copy(rp_h.at[relv[b]], rpv[b], semg[b])

    def _wait_g(b):
        pltpu.make_async_copy(hp_h.at[subv[b]], hpv[b], semg[b]).wait()
        pltpu.make_async_copy(rp_h.at[relv[b]], rpv[b], semg[b]).wait()

    eiota = lax.iota(jnp.int32, 16)

    def _alpha_chunk(b):
        def _egrp(v, carry):
            sgrp = jnp.zeros((16,), jnp.float32)
            for e0 in range(16):
                e = 16 * v + e0
                rid_e = ridv[b][pl.ds(e, 16)][0]
                acc = jnp.zeros((16,), jnp.float32)
                for j in range(A // 16):
                    sl = pl.ds(16 * j, 16)
                    pre = hpv[b][e, sl] + rpv[b][e, sl] + qp_v[rid_e, sl]
                    acc = acc + jnp.maximum(pre, 0.0) * wa_v[sl]
                for sh in (8, 4, 2, 1):
                    acc = acc + acc.at[eiota ^ sh].get(
                        mode="promise_in_bounds")
                sgrp = jnp.where(eiota == e0, acc, sgrp)
            x = sgrp + wb_v[...]
            alv[b][pl.ds(16 * v, 16)] = 1.0 / (1.0 + jnp.exp(-x))
            return carry

        lax.fori_loop(0, CH1 // 16, _egrp, 0)

    _issue(0, 0)

    def _pair(gp, carry):
        for b in range(2):
            g = 2 * gp + b

            @pl.when(g + 1 < NCH1)
            def _():
                _issue(g + 1, 1 - b)

            _wait_g(b)
            _alpha_chunk(b)
            pltpu.sync_copy(alv[b], al_h.at[pl.ds(tbase + g * CH1, CH1)])
        return carry

    lax.fori_loop(0, NCH1 // 2, _pair, 0)


def _sc_alpha_call(sub, rel, rid, hp, rp, qp, wa, wb16):
    mesh = plsc.VectorSubcoreMesh(core_axis_name="c", subcore_axis_name="s",
                                  num_cores=NC, num_subcores=NS)
    f = pl.kernel(
        _sc_alpha,
        out_type=jax.ShapeDtypeStruct((EPMAX,), jnp.float32),
        mesh=mesh,
        scratch_types=[
            pltpu.VMEM((CH1,), jnp.int32),
            pltpu.VMEM((CH1,), jnp.int32),
            pltpu.VMEM((CH1,), jnp.int32),
            pltpu.VMEM((CH1,), jnp.int32),
            pltpu.VMEM((CH1 + 16,), jnp.int32),
            pltpu.VMEM((CH1 + 16,), jnp.int32),
            pltpu.VMEM((CH1, A), jnp.float32),
            pltpu.VMEM((CH1, A), jnp.float32),
            pltpu.VMEM((CH1, A), jnp.float32),
            pltpu.VMEM((CH1, A), jnp.float32),
            pltpu.VMEM((B, A), jnp.float32),
            pltpu.VMEM((CH1,), jnp.float32),
            pltpu.VMEM((CH1,), jnp.float32),
            pltpu.VMEM((A,), jnp.float32),
            pltpu.VMEM((16,), jnp.float32),
            pltpu.SemaphoreType.DMA,
            pltpu.SemaphoreType.DMA,
        ],
    )
    return f(sub, rel, rid, hp, rp, qp, wa, wb16)


def _sc_msg(sub_h, rel_h, obj_h, al_h, hid_h, rla_h, out_h,
            suba_v, subb_v, rela_v, relb_v, ala_v, alb_v,
            obj0_v, obj1_v, obj2_v, obj3_v,
            hsa_v, hsb_v, hra_v, hrb_v, msga_v, msgb_v,
            agg_sh, semia, semib, semga, semgb, sems):
    c = lax.axis_index("c")
    s = lax.axis_index("s")
    tbase = s * EPT2

    subv = (suba_v, subb_v)
    relv = (rela_v, relb_v)
    alv = (ala_v, alb_v)
    objv = (obj0_v, obj1_v, obj2_v, obj3_v)
    hsv = (hsa_v, hsb_v)
    hrv = (hra_v, hrb_v)
    msgv = (msga_v, msgb_v)
    semi = (semia, semib)
    semg = (semga, semgb)

    z16 = jnp.zeros((16,), jnp.float32)

    def _zrow(i, carry):
        for j in range(DH // 16):
            msga_v[i, 16 * j:16 * (j + 1)] = z16
        return carry

    lax.fori_loop(0, CH2, _zrow, 0)
    nfull = STRIPE // CH2
    rem = STRIPE - nfull * CH2
    for k in range(nfull):
        pltpu.sync_copy(msga_v, agg_sh.at[pl.ds(s * STRIPE + k * CH2, CH2)])
    if rem:
        pltpu.sync_copy(msga_v.at[pl.ds(0, rem)],
                        agg_sh.at[pl.ds(s * STRIPE + nfull * CH2, rem)])
    plsc.subcore_barrier()

    def _issue_idx(g, b2, b4):
        base = tbase + g * CH2
        pltpu.async_copy(sub_h.at[pl.ds(base, CH2)], subv[b2], semi[b2])
        pltpu.async_copy(rel_h.at[pl.ds(base, CH2)], relv[b2], semi[b2])
        pltpu.async_copy(al_h.at[pl.ds(base, CH2)],
                         alv[b2].at[pl.ds(0, CH2)], semi[b2])
        pltpu.async_copy(obj_h.at[pl.ds(base, CH2)], objv[b4], semi[b2])

    def _wait_idx(b2, b4):
        pltpu.make_async_copy(sub_h.at[pl.ds(tbase, CH2)], subv[b2],
                              semi[b2]).wait()
        pltpu.make_async_copy(rel_h.at[pl.ds(tbase, CH2)], relv[b2],
                              semi[b2]).wait()
        pltpu.make_async_copy(al_h.at[pl.ds(tbase, CH2)],
                              alv[b2].at[pl.ds(0, CH2)], semi[b2]).wait()
        pltpu.make_async_copy(obj_h.at[pl.ds(tbase, CH2)], objv[b4],
                              semi[b2]).wait()

    def _issue_g(b2):
        pltpu.async_copy(hid_h.at[c].at[subv[b2]], hsv[b2], semg[b2])
        pltpu.async_copy(rla_h.at[c].at[relv[b2]], hrv[b2], semg[b2])

    def _wait_g(b2):
        pltpu.make_async_copy(hid_h.at[c].at[subv[b2]], hsv[b2],
                              semg[b2]).wait()
        pltpu.make_async_copy(rla_h.at[c].at[relv[b2]], hrv[b2],
                              semg[b2]).wait()

    def _wait_s(b2, b4):
        pltpu.make_async_copy(msgv[b2], agg_sh.at[objv[b4]], sems).wait()

    _issue_idx(0, 0, 0)
    _issue_idx(1, 1, 1)
    _wait_idx(0, 0)
    _issue_g(0)

    def _quad(qp, carry):
        for b in range(4):
            g = 4 * qp + b
            b2 = b % 2

            @pl.when(g >= 2)
            def _():
                _wait_s(b2, b)

            _wait_g(b2)

            @pl.when(g + 1 < NCH2)
            def _():
                _wait_idx(1 - b2, (b + 1) % 4)
                _issue_g(1 - b2)

            def _emsg(e, carry2):
                a = alv[b2][pl.ds(e, 16)][0]
                for j in range(DH // 16):
                    sl = pl.ds(16 * j, 16)
                    msgv[b2][e, sl] = hsv[b2][e, sl] * hrv[b2][e, sl] * a
                return carry2

            lax.fori_loop(0, CH2, _emsg, 0)

            @pl.when(g + 2 < NCH2)
            def _():
                _issue_idx(g + 2, b2, (b + 2) % 4)

            pltpu.async_copy(msgv[b2], agg_sh.at[objv[b]], sems, add=True)
        return carry

    lax.fori_loop(0, NCH2 // 4, _quad, 0)
    _wait_s(0, 2)
    _wait_s(1, 3)

    plsc.subcore_barrier()
    pltpu.sync_copy(agg_sh.at[pl.ds(s * STRIPE, STRIPE)],
                    out_h.at[c, pl.ds(s * STRIPE, STRIPE)])


def _sc_msg_call(sub, rel, obj, alpha, hid2, rla2):
    mesh = plsc.VectorSubcoreMesh(core_axis_name="c", subcore_axis_name="s",
                                  num_cores=NC, num_subcores=NS)
    f = pl.kernel(
        _sc_msg,
        out_type=jax.ShapeDtypeStruct((NC, NP, DH), jnp.float32),
        mesh=mesh,
        scratch_types=[
            pltpu.VMEM((CH2,), jnp.int32),
            pltpu.VMEM((CH2,), jnp.int32),
            pltpu.VMEM((CH2,), jnp.int32),
            pltpu.VMEM((CH2,), jnp.int32),
            pltpu.VMEM((CH2 + 16,), jnp.float32),
            pltpu.VMEM((CH2 + 16,), jnp.float32),
            pltpu.VMEM((CH2,), jnp.int32),
            pltpu.VMEM((CH2,), jnp.int32),
            pltpu.VMEM((CH2,), jnp.int32),
            pltpu.VMEM((CH2,), jnp.int32),
            pltpu.VMEM((CH2, DH), jnp.float32),
            pltpu.VMEM((CH2, DH), jnp.float32),
            pltpu.VMEM((CH2, DH), jnp.float32),
            pltpu.VMEM((CH2, DH), jnp.float32),
            pltpu.VMEM((CH2, DH), jnp.float32),
            pltpu.VMEM((CH2, DH), jnp.float32),
            pltpu.VMEM_SHARED((NP2, DH), jnp.float32),
            pltpu.SemaphoreType.DMA,
            pltpu.SemaphoreType.DMA,
            pltpu.SemaphoreType.DMA,
            pltpu.SemaphoreType.DMA,
            pltpu.SemaphoreType.DMA,
        ],
    )
    return f(sub, rel, obj, alpha, hid2, rla2)


def kernel(q_sub, q_rel, r_idx, hidden, edges, n_node, rela_embed,
           Ws_attn, Wr_attn, Wqr_attn_W, Wqr_attn_b,
           w_alpha_W, w_alpha_b, W_h):
    sub = jnp.pad(edges[:, 0].astype(jnp.int32), (0, EPMAX - E))
    rel = jnp.pad(edges[:, 1].astype(jnp.int32), (0, EPMAX - E))
    obj = jnp.pad(edges[:, 2].astype(jnp.int32), (0, EPMAX - E),
                  constant_values=NP2 - 1)
    rid = jnp.pad(r_idx.astype(jnp.int32), (0, EPMAX - E))

    hid_p = jnp.pad(hidden, ((0, NP - N), (0, 0)))
    rla_p = jnp.pad(rela_embed, ((0, NP - rela_embed.shape[0]), (0, 0)))

    hp = _matmul(hid_p, Ws_attn, 512)
    rp = _matmul(rla_p, Wr_attn, 512)
    qsel = jnp.take(rela_embed, q_rel, axis=0)
    qp = _matmul_bias(qsel, Wqr_attn_W, Wqr_attn_b)

    wa = w_alpha_W[:, 0]
    wb16 = jnp.broadcast_to(w_alpha_b, (16,)).astype(jnp.float32)

    alpha = _sc_alpha_call(sub, rel, rid, hp, rp, qp, wa, wb16)

    hid2 = hid_p.reshape(NP, NC, DH).transpose(1, 0, 2)
    rla2 = rla_p.reshape(NP, NC, DH).transpose(1, 0, 2)

    agg2 = _sc_msg_call(sub, rel, obj, alpha, hid2, rla2)

    w2 = W_h.reshape(NC, DH, D)
    out = _final_matmul(agg2, w2, 512)
    return out[:N]


# speedup vs baseline: 1.6365x; 1.2034x over previous
import jax
import jax.numpy as jnp
from jax import lax
from jax.experimental import pallas as pl
from jax.experimental.pallas import tpu as pltpu
from jax.experimental.pallas import tpu_sc as plsc

N = 10000
E = 160000
D = 256
A = 128
B = 256
NP = 10240
NC = 2
NS = 16
DH = D // NC

NP2 = 10048
STRIPE = 632
STRIPE_LAST = NP2 - (NS - 1) * STRIPE

EPMAX = 163840
CH1 = 128
EPP1 = EPMAX // (NC * NS)
NCH1 = EPP1 // CH1
CH2 = 64
EPT2 = EPMAX // NS
NCH2 = EPT2 // CH2


def _mm_body(x_ref, w_ref, o_ref):
    o_ref[...] = jnp.dot(x_ref[...], w_ref[...],
                         preferred_element_type=jnp.float32)


def _matmul(x, w, bm):
    m, k = x.shape
    n = w.shape[1]
    return pl.pallas_call(
        _mm_body,
        grid=(m // bm,),
        in_specs=[
            pl.BlockSpec((bm, k), lambda i: (i, 0)),
            pl.BlockSpec((k, n), lambda i: (0, 0)),
        ],
        out_specs=pl.BlockSpec((bm, n), lambda i: (i, 0)),
        out_shape=jax.ShapeDtypeStruct((m, n), jnp.float32),
    )(x, w)


def _mm_bias_body(x_ref, w_ref, b_ref, o_ref):
    o_ref[...] = jnp.dot(x_ref[...], w_ref[...],
                         preferred_element_type=jnp.float32) + b_ref[...]


def _matmul_bias(x, w, b):
    m, k = x.shape
    n = w.shape[1]
    return pl.pallas_call(
        _mm_bias_body,
        grid=(1,),
        in_specs=[
            pl.BlockSpec((m, k), lambda i: (0, 0)),
            pl.BlockSpec((k, n), lambda i: (0, 0)),
            pl.BlockSpec((1, n), lambda i: (0, 0)),
        ],
        out_specs=pl.BlockSpec((m, n), lambda i: (0, 0)),
        out_shape=jax.ShapeDtypeStruct((m, n), jnp.float32),
    )(x, w, b.reshape(1, n))


def _final_body(agg_ref, w2_ref, o_ref):
    a = agg_ref[...]
    w2 = w2_ref[...]
    o_ref[...] = (jnp.dot(a[0], w2[0], preferred_element_type=jnp.float32)
                  + jnp.dot(a[1], w2[1], preferred_element_type=jnp.float32))


def _final_matmul(agg2, w2, bm):
    m = agg2.shape[1]
    return pl.pallas_call(
        _final_body,
        grid=(m // bm,),
        in_specs=[
            pl.BlockSpec((NC, bm, DH), lambda i: (0, i, 0)),
            pl.BlockSpec((NC, DH, D), lambda i: (0, 0, 0)),
        ],
        out_specs=pl.BlockSpec((bm, D), lambda i: (i, 0)),
        out_shape=jax.ShapeDtypeStruct((m, D), jnp.float32),
    )(agg2, w2)


def _alpha_groups(ngrp, hp_v, rp_v, qp_v, wa_v, wb_v, al_v):
    eiota = lax.iota(jnp.int32, 16)

    def _egrp(v, carry):
        sgrp = jnp.zeros((16,), jnp.float32)
        for e0 in range(16):
            e = 16 * v + e0
            acc = jnp.zeros((16,), jnp.float32)
            for j in range(A // 16):
                sl = pl.ds(16 * j, 16)
                pre = hp_v[e, sl] + rp_v[e, sl] + qp_v[e, sl]
                acc = acc + jnp.maximum(pre, 0.0) * wa_v[sl]
            for sh in (8, 4, 2, 1):
                acc = acc + acc.at[eiota ^ sh].get(mode="promise_in_bounds")
            sgrp = jnp.where(eiota == e0, acc, sgrp)
        x = sgrp + wb_v[...]
        al_v[pl.ds(16 * v, 16)] = 1.0 / (1.0 + jnp.exp(-x))
        return carry

    lax.fori_loop(0, ngrp, _egrp, 0)


def _sc_alpha(sub_h, rel_h, rid_h, hp_h, rp_h, qp_h, wa_h, wb_h, al_h,
              suba_v, subb_v, rela_v, relb_v, rida_v, ridb_v,
              hpa_v, hpb_v, rpa_v, rpb_v,
              qp_v, ala_v, alb_v, wa_v, wb_v, semga, semgb):
    c = lax.axis_index("c")
    s = lax.axis_index("s")
    w = s * NC + c
    tbase = w * EPP1

    pltpu.sync_copy(wa_h, wa_v)
    pltpu.sync_copy(wb_h, wb_v)
    pltpu.sync_copy(qp_h, qp_v)

    subv = (suba_v, subb_v)
    relv = (rela_v, relb_v)
    ridv = (rida_v, ridb_v)
    hpv = (hpa_v, hpb_v)
    rpv = (rpa_v, rpb_v)
    alv = (ala_v, alb_v)
    semg = (semga, semgb)

    def _issue(g, b):
        base = tbase + g * CH1
        pltpu.sync_copy(sub_h.at[pl.ds(base, CH1)], subv[b])
        pltpu.sync_copy(rel_h.at[pl.ds(base, CH1)], relv[b])
        pltpu.sync_copy(rid_h.at[pl.ds(base, CH1)],
                        ridv[b].at[pl.ds(0, CH1)])
        pltpu.async_copy(hp_h.at[subv[b]], hpv[b], semg[b])
        pltpu.async_copy(rp_h.at[relv[b]], rpv[b], semg[b])

    def _wait_g(b):
        pltpu.make_async_copy(hp_h.at[subv[b]], hpv[b], semg[b]).wait()
        pltpu.make_async_copy(rp_h.at[relv[b]], rpv[b], semg[b]).wait()

    eiota = lax.iota(jnp.int32, 16)

    def _alpha_chunk(b):
        def _egrp(v, carry):
            sgrp = jnp.zeros((16,), jnp.float32)
            for e0 in range(16):
                e = 16 * v + e0
                rid_e = ridv[b][pl.ds(e, 16)][0]
                acc = jnp.zeros((16,), jnp.float32)
                for j in range(A // 16):
                    sl = pl.ds(16 * j, 16)
                    pre = hpv[b][e, sl] + rpv[b][e, sl] + qp_v[rid_e, sl]
                    acc = acc + jnp.maximum(pre, 0.0) * wa_v[sl]
                for sh in (8, 4, 2, 1):
                    acc = acc + acc.at[eiota ^ sh].get(
                        mode="promise_in_bounds")
                sgrp = jnp.where(eiota == e0, acc, sgrp)
            x = sgrp + wb_v[...]
            alv[b][pl.ds(16 * v, 16)] = 1.0 / (1.0 + jnp.exp(-x))
            return carry

        lax.fori_loop(0, CH1 // 16, _egrp, 0)

    _issue(0, 0)

    def _pair(gp, carry):
        for b in range(2):
            g = 2 * gp + b

            @pl.when(g + 1 < NCH1)
            def _():
                _issue(g + 1, 1 - b)

            _wait_g(b)
            _alpha_chunk(b)
            pltpu.sync_copy(alv[b], al_h.at[pl.ds(tbase + g * CH1, CH1)])
        return carry

    lax.fori_loop(0, NCH1 // 2, _pair, 0)


def _sc_alpha_call(sub, rel, rid, hp, rp, qp, wa, wb16):
    mesh = plsc.VectorSubcoreMesh(core_axis_name="c", subcore_axis_name="s",
                                  num_cores=NC, num_subcores=NS)
    f = pl.kernel(
        _sc_alpha,
        out_type=jax.ShapeDtypeStruct((EPMAX,), jnp.float32),
        mesh=mesh,
        scratch_types=[
            pltpu.VMEM((CH1,), jnp.int32),
            pltpu.VMEM((CH1,), jnp.int32),
            pltpu.VMEM((CH1,), jnp.int32),
            pltpu.VMEM((CH1,), jnp.int32),
            pltpu.VMEM((CH1 + 16,), jnp.int32),
            pltpu.VMEM((CH1 + 16,), jnp.int32),
            pltpu.VMEM((CH1, A), jnp.float32),
            pltpu.VMEM((CH1, A), jnp.float32),
            pltpu.VMEM((CH1, A), jnp.float32),
            pltpu.VMEM((CH1, A), jnp.float32),
            pltpu.VMEM((B, A), jnp.float32),
            pltpu.VMEM((CH1,), jnp.float32),
            pltpu.VMEM((CH1,), jnp.float32),
            pltpu.VMEM((A,), jnp.float32),
            pltpu.VMEM((16,), jnp.float32),
            pltpu.SemaphoreType.DMA,
            pltpu.SemaphoreType.DMA,
        ],
    )
    return f(sub, rel, rid, hp, rp, qp, wa, wb16)


def _sc_msg(sub_h, rel_h, obj_h, al_h, hid_h, rla_h, out_h,
            suba_v, subb_v, rela_v, relb_v, ala_v, alb_v,
            obj0_v, obj1_v, obj2_v, obj3_v,
            hsa_v, hsb_v, hra_v, hrb_v, msga_v, msgb_v,
            agg_sh, semia, semib, semga, semgb, sems):
    c = lax.axis_index("c")
    s = lax.axis_index("s")
    tbase = s * EPT2

    subv = (suba_v, subb_v)
    relv = (rela_v, relb_v)
    alv = (ala_v, alb_v)
    objv = (obj0_v, obj1_v, obj2_v, obj3_v)
    hsv = (hsa_v, hsb_v)
    hrv = (hra_v, hrb_v)
    msgv = (msga_v, msgb_v)
    semi = (semia, semib)
    semg = (semga, semgb)

    z16 = jnp.zeros((16,), jnp.float32)

    def _zrow(i, carry):
        for j in range(DH // 16):
            msga_v[i, 16 * j:16 * (j + 1)] = z16
        return carry

    lax.fori_loop(0, CH2, _zrow, 0)
    nfull = STRIPE_LAST // CH2
    rem = STRIPE_LAST - nfull * CH2
    for k in range(nfull):
        pltpu.sync_copy(msga_v, agg_sh.at[pl.ds(s * STRIPE + k * CH2, CH2)])
    if rem:
        pltpu.sync_copy(msga_v.at[pl.ds(0, rem)],
                        agg_sh.at[pl.ds(s * STRIPE + nfull * CH2, rem)])

    @pl.when(s < NS - 1)
    def _():
        pltpu.sync_copy(msga_v,
                        agg_sh.at[pl.ds(s * STRIPE + STRIPE_LAST, CH2)])
    plsc.subcore_barrier()

    def _issue_idx(g, b2, b4):
        base = tbase + g * CH2
        pltpu.async_copy(sub_h.at[pl.ds(base, CH2)], subv[b2], semi[b2])
        pltpu.async_copy(rel_h.at[pl.ds(base, CH2)], relv[b2], semi[b2])
        pltpu.async_copy(al_h.at[pl.ds(base, CH2)],
                         alv[b2].at[pl.ds(0, CH2)], semi[b2])
        pltpu.async_copy(obj_h.at[pl.ds(base, CH2)], objv[b4], semi[b2])

    def _wait_idx(b2, b4):
        pltpu.make_async_copy(sub_h.at[pl.ds(tbase, CH2)], subv[b2],
                              semi[b2]).wait()
        pltpu.make_async_copy(rel_h.at[pl.ds(tbase, CH2)], relv[b2],
                              semi[b2]).wait()
        pltpu.make_async_copy(al_h.at[pl.ds(tbase, CH2)],
                              alv[b2].at[pl.ds(0, CH2)], semi[b2]).wait()
        pltpu.make_async_copy(obj_h.at[pl.ds(tbase, CH2)], objv[b4],
                              semi[b2]).wait()

    def _issue_g(b2):
        pltpu.async_copy(hid_h.at[c].at[subv[b2]], hsv[b2], semg[b2])
        pltpu.async_copy(rla_h.at[c].at[relv[b2]], hrv[b2], semg[b2])

    def _wait_g(b2):
        pltpu.make_async_copy(hid_h.at[c].at[subv[b2]], hsv[b2],
                              semg[b2]).wait()
        pltpu.make_async_copy(rla_h.at[c].at[relv[b2]], hrv[b2],
                              semg[b2]).wait()

    def _wait_s(b2, b4):
        pltpu.make_async_copy(msgv[b2], agg_sh.at[objv[b4]], sems).wait()

    _issue_idx(0, 0, 0)
    _issue_idx(1, 1, 1)
    _wait_idx(0, 0)
    _issue_g(0)

    def _quad(qp, carry):
        for b in range(4):
            g = 4 * qp + b
            b2 = b % 2

            @pl.when(g >= 2)
            def _():
                _wait_s(b2, b)

            _wait_g(b2)

            @pl.when(g + 1 < NCH2)
            def _():
                _wait_idx(1 - b2, (b + 1) % 4)
                _issue_g(1 - b2)

            def _emsg(e, carry2):
                a = alv[b2][pl.ds(e, 16)][0]
                for j in range(DH // 16):
                    sl = pl.ds(16 * j, 16)
                    msgv[b2][e, sl] = hsv[b2][e, sl] * hrv[b2][e, sl] * a
                return carry2

            lax.fori_loop(0, CH2, _emsg, 0)

            @pl.when(g + 2 < NCH2)
            def _():
                _issue_idx(g + 2, b2, (b + 2) % 4)

            pltpu.async_copy(msgv[b2], agg_sh.at[objv[b]], sems, add=True)
        return carry

    lax.fori_loop(0, NCH2 // 4, _quad, 0)
    _wait_s(0, 2)
    _wait_s(1, 3)

    plsc.subcore_barrier()
    pltpu.sync_copy(agg_sh.at[pl.ds(s * STRIPE, STRIPE_LAST)],
                    out_h.at[c, pl.ds(s * STRIPE, STRIPE_LAST)])

    @pl.when(s < NS - 1)
    def _():
        pltpu.sync_copy(
            agg_sh.at[pl.ds(s * STRIPE + STRIPE_LAST, STRIPE - STRIPE_LAST)],
            out_h.at[c, pl.ds(s * STRIPE + STRIPE_LAST,
                              STRIPE - STRIPE_LAST)])


def _sc_msg_call(sub, rel, obj, alpha, hid2, rla2):
    mesh = plsc.VectorSubcoreMesh(core_axis_name="c", subcore_axis_name="s",
                                  num_cores=NC, num_subcores=NS)
    f = pl.kernel(
        _sc_msg,
        out_type=jax.ShapeDtypeStruct((NC, NP, DH), jnp.float32),
        mesh=mesh,
        scratch_types=[
            pltpu.VMEM((CH2,), jnp.int32),
            pltpu.VMEM((CH2,), jnp.int32),
            pltpu.VMEM((CH2,), jnp.int32),
            pltpu.VMEM((CH2,), jnp.int32),
            pltpu.VMEM((CH2 + 16,), jnp.float32),
            pltpu.VMEM((CH2 + 16,), jnp.float32),
            pltpu.VMEM((CH2,), jnp.int32),
            pltpu.VMEM((CH2,), jnp.int32),
            pltpu.VMEM((CH2,), jnp.int32),
            pltpu.VMEM((CH2,), jnp.int32),
            pltpu.VMEM((CH2, DH), jnp.float32),
            pltpu.VMEM((CH2, DH), jnp.float32),
            pltpu.VMEM((CH2, DH), jnp.float32),
            pltpu.VMEM((CH2, DH), jnp.float32),
            pltpu.VMEM((CH2, DH), jnp.float32),
            pltpu.VMEM((CH2, DH), jnp.float32),
            pltpu.VMEM_SHARED((NP2, DH), jnp.float32),
            pltpu.SemaphoreType.DMA,
            pltpu.SemaphoreType.DMA,
            pltpu.SemaphoreType.DMA,
            pltpu.SemaphoreType.DMA,
            pltpu.SemaphoreType.DMA,
        ],
    )
    return f(sub, rel, obj, alpha, hid2, rla2)


def kernel(q_sub, q_rel, r_idx, hidden, edges, n_node, rela_embed,
           Ws_attn, Wr_attn, Wqr_attn_W, Wqr_attn_b,
           w_alpha_W, w_alpha_b, W_h):
    sub = jnp.pad(edges[:, 0].astype(jnp.int32), (0, EPMAX - E))
    rel = jnp.pad(edges[:, 1].astype(jnp.int32), (0, EPMAX - E))
    obj = jnp.pad(edges[:, 2].astype(jnp.int32), (0, EPMAX - E),
                  constant_values=NP2 - 1)
    rid = jnp.pad(r_idx.astype(jnp.int32), (0, EPMAX - E))

    hid_p = jnp.pad(hidden, ((0, NP - N), (0, 0)))
    rla_p = jnp.pad(rela_embed, ((0, NP - rela_embed.shape[0]), (0, 0)))

    hp = _matmul(hid_p, Ws_attn, 512)
    rp = _matmul(rla_p, Wr_attn, 512)
    qsel = jnp.take(rela_embed, q_rel, axis=0)
    qp = _matmul_bias(qsel, Wqr_attn_W, Wqr_attn_b)

    wa = w_alpha_W[:, 0]
    wb16 = jnp.broadcast_to(w_alpha_b, (16,)).astype(jnp.float32)

    alpha = _sc_alpha_call(sub, rel, rid, hp, rp, qp, wa, wb16)

    hid2 = hid_p.reshape(NP, NC, DH).transpose(1, 0, 2)
    rla2 = rla_p.reshape(NP, NC, DH).transpose(1, 0, 2)

    agg2 = _sc_msg_call(sub, rel, obj, alpha, hid2, rla2)

    w2 = W_h.reshape(NC, DH, D)
    out = _final_matmul(agg2, w2, 512)
    return out[:N]
